# baseline (speedup 1.0000x reference)
import functools

import jax
import jax.numpy as jnp
from jax import lax
from jax.experimental import pallas as pl
from jax.experimental.pallas import tpu as pltpu
from jax.experimental.pallas import tpu_sc as plsc

_N_ROWS, _N_COLS = 16384, 1024
_N = _N_ROWS * _N_COLS
_NW = 32
_PER_W = _N // _NW
_CHUNK = 16384
_N_CHUNKS = _PER_W // _CHUNK


def _cell_tables(g_table):
    xp = jnp.minimum(jnp.arange(9) + 1, 8)
    yp = jnp.minimum(jnp.arange(5) + 1, 4)
    t00 = g_table
    t01 = g_table[:, xp]
    t10 = g_table[yp, :]
    t11 = t10[:, xp]
    v, gx, gy, gxy = t00, t01 - t00, t10 - t00, t11 - t10 - t01 + t00

    def pad(a):
        return jnp.zeros((8, 16), jnp.float32).at[:5, :9].set(a).reshape(128)

    return pad(v), pad(gx), pad(gy), pad(gxy)


def _sc_body(v_h, gx_h, gy_h, gxy_h, ibmin_h, ysc_h, phi_h, sq_h, out_h,
             v_t, gx_t, gy_t, gxy_t, ibmin_t, ysc_t,
             phi_v0, phi_v1, sq_v0, sq_v1, out_v0, out_v1,
             sp0, sp1, sqs0, sqs1, so0, so1):
    nc = 2
    wid = lax.axis_index("s") * nc + lax.axis_index("c")
    pltpu.sync_copy(v_h, v_t)
    pltpu.sync_copy(gx_h, gx_t)
    pltpu.sync_copy(gy_h, gy_t)
    pltpu.sync_copy(gxy_h, gxy_t)
    pltpu.sync_copy(ibmin_h, ibmin_t)
    pltpu.sync_copy(ysc_h, ysc_t)
    ib_min = ibmin_t[...]
    yscale = ysc_t[...]
    base = wid * _PER_W
    sps = (sp0, sp1)
    sqs = (sqs0, sqs1)
    sos = (so0, so1)
    phi_v = (phi_v0, phi_v1)
    sq_v = (sq_v0, sq_v1)
    out_v = (out_v0, out_v1)

    def in_phi(ci, b):
        off = base + ci * _CHUNK
        return pltpu.make_async_copy(
            phi_h.at[pl.ds(off, _CHUNK)], phi_v[b], sps[b])

    def in_sq(ci, b):
        off = base + ci * _CHUNK
        return pltpu.make_async_copy(
            sq_h.at[pl.ds(off, _CHUNK)], sq_v[b], sqs[b])

    def out_dma(ci, b):
        off = base + ci * _CHUNK
        return pltpu.make_async_copy(
            out_v[b], out_h.at[pl.ds(off, _CHUNK)], sos[b])

    in_phi(0, 0).start()
    in_sq(0, 0).start()
    in_phi(1, 1).start()
    in_sq(1, 1).start()

    def outer(i, carry):
        for b in range(2):
            ci = i * 2 + b
            in_phi(ci, b).wait()
            in_sq(ci, b).wait()

            @pl.when(i > 0)
            def _wait_out():
                out_dma(ci - 2, b).wait()

            pv = phi_v[b]
            qv = sq_v[b]
            ov = out_v[b]

            @plsc.parallel_loop(0, _CHUNK, step=16, unroll=2)
            def vec_body(o):
                p = pv[pl.ds(o, 16)]
                s = qv[pl.ds(o, 16)]
                a = jnp.abs(lax.rem(p, 1.0))
                ax = a * 16.0
                x = jnp.minimum(ax, 16.0 - ax)
                y = jnp.clip((s - ib_min) * yscale, 0.0, 4.0)
                x0 = x.astype(jnp.int32)
                y0 = y.astype(jnp.int32)
                fx = x - x0.astype(jnp.float32)
                fy = y - y0.astype(jnp.float32)
                idx = lax.shift_left(y0, 4) + x0
                g0 = plsc.load_gather(v_t, [idx])
                g1 = plsc.load_gather(gx_t, [idx])
                g2 = plsc.load_gather(gy_t, [idx])
                g3 = plsc.load_gather(gxy_t, [idx])
                ov[pl.ds(o, 16)] = (g0 + fx * g1) + fy * (g2 + fx * g3)

            out_dma(ci, b).start()

            @pl.when(ci + 2 < _N_CHUNKS)
            def _next_in():
                in_phi(ci + 2, b).start()
                in_sq(ci + 2, b).start()
        return carry

    lax.fori_loop(0, _N_CHUNKS // 2, outer, 0)
    out_dma(_N_CHUNKS - 2, 0).wait()
    out_dma(_N_CHUNKS - 1, 1).wait()


def kernel(phi, squid_current, g_table, ib_list):
    v, gx, gy, gxy = _cell_tables(g_table)
    ibmin16 = jnp.full((16,), ib_list[0], jnp.float32)
    ysc16 = jnp.full((16,), 4.0 / (ib_list[-1] - ib_list[0]), jnp.float32)
    mesh = plsc.VectorSubcoreMesh(core_axis_name="c", subcore_axis_name="s")
    kern = functools.partial(
        pl.kernel,
        mesh=mesh,
        compiler_params=pltpu.CompilerParams(needs_layout_passes=False),
        out_type=jax.ShapeDtypeStruct((_N,), jnp.float32),
        scratch_types=[
            pltpu.VMEM((128,), jnp.float32),
            pltpu.VMEM((128,), jnp.float32),
            pltpu.VMEM((128,), jnp.float32),
            pltpu.VMEM((128,), jnp.float32),
            pltpu.VMEM((16,), jnp.float32),
            pltpu.VMEM((16,), jnp.float32),
            pltpu.VMEM((_CHUNK,), jnp.float32),
            pltpu.VMEM((_CHUNK,), jnp.float32),
            pltpu.VMEM((_CHUNK,), jnp.float32),
            pltpu.VMEM((_CHUNK,), jnp.float32),
            pltpu.VMEM((_CHUNK,), jnp.float32),
            pltpu.VMEM((_CHUNK,), jnp.float32),
            pltpu.SemaphoreType.DMA,
            pltpu.SemaphoreType.DMA,
            pltpu.SemaphoreType.DMA,
            pltpu.SemaphoreType.DMA,
            pltpu.SemaphoreType.DMA,
            pltpu.SemaphoreType.DMA,
        ],
    )(_sc_body)
    out = kern(v, gx, gy, gxy, ibmin16, ysc16,
               phi.reshape(_N), squid_current.reshape(_N))
    return out.reshape(_N_ROWS, _N_COLS)


# speedup vs baseline: 1.2596x; 1.2596x over previous
import functools

import jax
import jax.numpy as jnp
from jax import lax
from jax.experimental import pallas as pl
from jax.experimental.pallas import tpu as pltpu
from jax.experimental.pallas import tpu_sc as plsc

_N_ROWS, _N_COLS = 16384, 1024
_SC_ROWS = 6144
_TC_ROWS = _N_ROWS - _SC_ROWS
_NW = 32
_CHUNK = 16384
_N_SC = _SC_ROWS * _N_COLS
_PER_W = _N_SC // _NW
_N_CHUNKS = _PER_W // _CHUNK


def _cell_tables(g_table):
    xp = jnp.minimum(jnp.arange(9) + 1, 8)
    yp = jnp.minimum(jnp.arange(5) + 1, 4)
    t00 = g_table
    t01 = g_table[:, xp]
    t10 = g_table[yp, :]
    t11 = t10[:, xp]
    v, gx, gy, gxy = t00, t01 - t00, t10 - t00, t11 - t10 - t01 + t00

    def pad(a):
        return jnp.zeros((8, 16), jnp.float32).at[:5, :9].set(a).reshape(128)

    return pad(v), pad(gx), pad(gy), pad(gxy)


def _basis_coeffs(g_table):
    sx = jnp.diff(g_table, axis=1)
    cx = jnp.concatenate([g_table[:, :1], sx[:, :1], jnp.diff(sx, axis=1)], axis=1)
    sy = jnp.diff(cx, axis=0)
    return jnp.concatenate([cx[:1], sy[:1], jnp.diff(sy, axis=0)], axis=0)


def _sc_body(v_h, gx_h, gy_h, gxy_h, ibmin_h, ysc_h, phi_h, sq_h, out_h,
             v_t, gx_t, gy_t, gxy_t, ibmin_t, ysc_t,
             phi_v0, phi_v1, sq_v0, sq_v1, out_v0, out_v1,
             sp0, sp1, sqs0, sqs1, so0, so1):
    nc = 2
    wid = lax.axis_index("s") * nc + lax.axis_index("c")
    pltpu.sync_copy(v_h, v_t)
    pltpu.sync_copy(gx_h, gx_t)
    pltpu.sync_copy(gy_h, gy_t)
    pltpu.sync_copy(gxy_h, gxy_t)
    pltpu.sync_copy(ibmin_h, ibmin_t)
    pltpu.sync_copy(ysc_h, ysc_t)
    ib_min = ibmin_t[...]
    yscale = ysc_t[...]
    base = wid * _PER_W
    sps = (sp0, sp1)
    sqs = (sqs0, sqs1)
    sos = (so0, so1)
    phi_v = (phi_v0, phi_v1)
    sq_v = (sq_v0, sq_v1)
    out_v = (out_v0, out_v1)

    def in_phi(ci, b):
        off = base + ci * _CHUNK
        return pltpu.make_async_copy(
            phi_h.at[pl.ds(off, _CHUNK)], phi_v[b], sps[b])

    def in_sq(ci, b):
        off = base + ci * _CHUNK
        return pltpu.make_async_copy(
            sq_h.at[pl.ds(off, _CHUNK)], sq_v[b], sqs[b])

    def out_dma(ci, b):
        off = base + ci * _CHUNK
        return pltpu.make_async_copy(
            out_v[b], out_h.at[pl.ds(off, _CHUNK)], sos[b])

    in_phi(0, 0).start()
    in_sq(0, 0).start()
    in_phi(1, 1).start()
    in_sq(1, 1).start()

    def outer(i, carry):
        for b in range(2):
            ci = i * 2 + b
            in_phi(ci, b).wait()
            in_sq(ci, b).wait()

            @pl.when(i > 0)
            def _wait_out():
                out_dma(ci - 2, b).wait()

            pv = phi_v[b]
            qv = sq_v[b]
            ov = out_v[b]

            @plsc.parallel_loop(0, _CHUNK, step=16, unroll=4)
            def vec_body(o):
                p = pv[pl.ds(o, 16)]
                s = qv[pl.ds(o, 16)]
                a = jnp.abs(lax.rem(p, 1.0))
                ax = a * 16.0
                x = jnp.minimum(ax, 16.0 - ax)
                y = jnp.clip((s - ib_min) * yscale, 0.0, 4.0)
                x0 = x.astype(jnp.int32)
                y0 = y.astype(jnp.int32)
                fx = x - x0.astype(jnp.float32)
                fy = y - y0.astype(jnp.float32)
                idx = lax.shift_left(y0, 4) + x0
                g0 = plsc.load_gather(v_t, [idx])
                g1 = plsc.load_gather(gx_t, [idx])
                g2 = plsc.load_gather(gy_t, [idx])
                g3 = plsc.load_gather(gxy_t, [idx])
                ov[pl.ds(o, 16)] = (g0 + fx * g1) + fy * (g2 + fx * g3)

            out_dma(ci, b).start()

            @pl.when(ci + 2 < _N_CHUNKS)
            def _next_in():
                in_phi(ci + 2, b).start()
                in_sq(ci + 2, b).start()
        return carry

    lax.fori_loop(0, _N_CHUNKS // 2, outer, 0)
    out_dma(_N_CHUNKS - 2, 0).wait()
    out_dma(_N_CHUNKS - 1, 1).wait()


def _tc_body(c_ref, pp_ref, phi_ref, sc_ref, o_ref):
    p = phi_ref[...]
    s = sc_ref[...]
    m = p - jnp.floor(p)
    pe = jnp.minimum(m, 1.0 - m)
    x = jnp.minimum(pe * 16.0, 8.0)
    y = jnp.clip((s - pp_ref[0, 0]) * pp_ref[0, 1], 0.0, 4.0)
    xb = [x] + [jnp.maximum(x - float(w), 0.0) for w in range(1, 8)]
    yb = [y] + [jnp.maximum(y - float(h), 0.0) for h in range(1, 4)]
    out = None
    for j in range(5):
        acc = c_ref[j, 0] + c_ref[j, 1] * xb[0]
        for i in range(2, 9):
            acc = acc + c_ref[j, i] * xb[i - 1]
        out = acc if j == 0 else out + yb[j - 1] * acc
    o_ref[...] = out


def kernel(phi, squid_current, g_table, ib_list):
    v, gx, gy, gxy = _cell_tables(g_table)
    coeffs = _basis_coeffs(g_table)
    ib_min = ib_list[0]
    yscale = 4.0 / (ib_list[-1] - ib_list[0])
    ibmin16 = jnp.full((16,), ib_min, jnp.float32)
    ysc16 = jnp.full((16,), yscale, jnp.float32)
    pp = jnp.stack([ib_min, yscale]).reshape(1, 2)

    mesh = plsc.VectorSubcoreMesh(core_axis_name="c", subcore_axis_name="s")
    sc_kern = functools.partial(
        pl.kernel,
        mesh=mesh,
        compiler_params=pltpu.CompilerParams(needs_layout_passes=False),
        out_type=jax.ShapeDtypeStruct((_N_SC,), jnp.float32),
        scratch_types=[
            pltpu.VMEM((128,), jnp.float32),
            pltpu.VMEM((128,), jnp.float32),
            pltpu.VMEM((128,), jnp.float32),
            pltpu.VMEM((128,), jnp.float32),
            pltpu.VMEM((16,), jnp.float32),
            pltpu.VMEM((16,), jnp.float32),
            pltpu.VMEM((_CHUNK,), jnp.float32),
            pltpu.VMEM((_CHUNK,), jnp.float32),
            pltpu.VMEM((_CHUNK,), jnp.float32),
            pltpu.VMEM((_CHUNK,), jnp.float32),
            pltpu.VMEM((_CHUNK,), jnp.float32),
            pltpu.VMEM((_CHUNK,), jnp.float32),
            pltpu.SemaphoreType.DMA,
            pltpu.SemaphoreType.DMA,
            pltpu.SemaphoreType.DMA,
            pltpu.SemaphoreType.DMA,
            pltpu.SemaphoreType.DMA,
            pltpu.SemaphoreType.DMA,
        ],
    )(_sc_body)

    out_sc = sc_kern(v, gx, gy, gxy, ibmin16, ysc16,
                     phi[:_SC_ROWS].reshape(_N_SC),
                     squid_current[:_SC_ROWS].reshape(_N_SC))

    block_rows = 256
    out_tc = pl.pallas_call(
        _tc_body,
        grid=(_TC_ROWS // block_rows,),
        in_specs=[
            pl.BlockSpec(memory_space=pltpu.SMEM),
            pl.BlockSpec(memory_space=pltpu.SMEM),
            pl.BlockSpec((block_rows, _N_COLS), lambda i: (i, 0)),
            pl.BlockSpec((block_rows, _N_COLS), lambda i: (i, 0)),
        ],
        out_specs=pl.BlockSpec((block_rows, _N_COLS), lambda i: (i, 0)),
        out_shape=jax.ShapeDtypeStruct((_TC_ROWS, _N_COLS), jnp.float32),
    )(coeffs, pp, phi[_SC_ROWS:], squid_current[_SC_ROWS:])

    return jnp.concatenate(
        [out_sc.reshape(_SC_ROWS, _N_COLS), out_tc], axis=0)
